# Initial kernel scaffold; baseline (speedup 1.0000x reference)
import jax
import jax.numpy as jnp
from jax.experimental import pallas as pl


def kernel(all_keypoint_batches, edge_index, conv1_W, conv1_b, conv2_W, conv2_b, W_ih, W_hh, b_ih, b_hh, fc_W, fc_b):
    raise NotImplementedError("write your pallas kernel here")


# speedup vs baseline: 44.5271x; 44.5271x over previous
import functools

import jax
import jax.numpy as jnp
from jax.experimental import pallas as pl
from jax.experimental.pallas import tpu as pltpu


def _fused_step(x_ref, e_ref, w1_ref, b1_ref, w2_ref, b2_ref,
                wih_ref, whh_ref, bias_ref, fcw_ref, fcb_ref,
                out_ref, h_ref, c_ref, m_ref, *, N, NF, B, H, LH, W, CH):
    w = pl.program_id(0)
    f32 = jnp.float32

    @pl.when(w == 0)
    def _init():
        h_ref[...] = jnp.zeros_like(h_ref)
        c_ref[...] = jnp.zeros_like(c_ref)
        m_ref[...] = jnp.full_like(m_ref, -jnp.inf)

    e = e_ref[...]
    erow = e[0:1, :]
    ecol = e[1:2, :]
    Epad = e.shape[1]
    nio = jax.lax.broadcasted_iota(jnp.int32, (N, Epad), 0).astype(f32)
    Oc = (ecol == nio).astype(f32)
    Or = (erow == nio).astype(f32)
    A = jax.lax.dot_general(Oc, Or, (((1,), (1,)), ((), ())),
                            preferred_element_type=f32)
    ri = jax.lax.broadcasted_iota(jnp.int32, (N, N), 0)
    ci = jax.lax.broadcasted_iota(jnp.int32, (N, N), 1)
    eye = (ri == ci).astype(f32)
    Ah = A + eye
    deg = jnp.sum(Ah, axis=1, keepdims=True)
    dinv = jax.lax.rsqrt(deg)
    D = eye * dinv
    An = jnp.dot(jnp.dot(D, Ah, preferred_element_type=f32), D,
                 preferred_element_type=f32)

    w1 = w1_ref[...]
    w2 = w2_ref[...]
    b1 = b1_ref[...]
    b2 = b2_ref[...]
    pool = jnp.full((1, N), 1.0 / N, dtype=f32)

    for t in range(CH):
        x = x_ref[t]
        xw1 = jnp.dot(x, w1, preferred_element_type=f32)
        t1 = jnp.dot(An, xw1.reshape(N, B * H), preferred_element_type=f32)
        h1 = jnp.maximum(t1.reshape(N * B, H) + b1, 0.0)
        xw2 = jnp.dot(h1, w2, preferred_element_type=f32)
        t2 = jnp.dot(An, xw2.reshape(N, B * H), preferred_element_type=f32)
        h2 = jnp.maximum(t2.reshape(N * B, H) + b2, 0.0)
        pooled = jnp.dot(pool, h2.reshape(N, B * H),
                         preferred_element_type=f32).reshape(B, H)

        gates = (jnp.dot(pooled, wih_ref[...], preferred_element_type=f32)
                 + jnp.dot(h_ref[...], whh_ref[...], preferred_element_type=f32)
                 + bias_ref[...])
        ig = jax.nn.sigmoid(gates[:, 0 * LH:1 * LH])
        fg = jax.nn.sigmoid(gates[:, 1 * LH:2 * LH])
        gg = jnp.tanh(gates[:, 2 * LH:3 * LH])
        og = jax.nn.sigmoid(gates[:, 3 * LH:4 * LH])
        c_new = fg * c_ref[...] + ig * gg
        h_new = og * jnp.tanh(c_new)
        c_ref[...] = c_new
        h_ref[...] = h_new
        m_ref[...] = jnp.maximum(m_ref[...], h_new)

    @pl.when(w == (W // CH) - 1)
    def _final():
        logits = jnp.dot(m_ref[...], fcw_ref[...],
                         preferred_element_type=f32) + fcb_ref[...]
        out_ref[...] = jax.nn.sigmoid(logits)


def kernel(all_keypoint_batches, edge_index, conv1_W, conv1_b, conv2_W,
           conv2_b, W_ih, W_hh, b_ih, b_hh, fc_W, fc_b):
    B, W, N, NF = all_keypoint_batches.shape
    H = conv1_W.shape[1]
    LH = W_hh.shape[1]
    E = edge_index.shape[1]
    CH = 1

    Xn = jnp.transpose(all_keypoint_batches, (1, 2, 0, 3)).reshape(W, N * B, NF)

    Epad = max(32, ((E + 31) // 32) * 32)
    epad = jnp.full((8, Epad), -1.0, dtype=jnp.float32)
    epad = epad.at[:2, :E].set(edge_index.astype(jnp.float32))

    wih = W_ih.T
    whh = W_hh.T
    bias = (b_ih + b_hh).reshape(1, 4 * LH)
    b1 = conv1_b.reshape(1, H)
    b2 = conv2_b.reshape(1, H)
    fcw = fc_W.T
    fcb = fc_b.reshape(1, fc_W.shape[0])

    full = lambda shape: pl.BlockSpec(shape, lambda w: (0,) * len(shape))
    grid = W // CH

    out = pl.pallas_call(
        functools.partial(_fused_step, N=N, NF=NF, B=B, H=H, LH=LH, W=W,
                          CH=CH),
        grid=(grid,),
        in_specs=[
            pl.BlockSpec((CH, N * B, NF), lambda w: (w, 0, 0)),
            full((8, Epad)),
            full((NF, H)),
            full((1, H)),
            full((H, H)),
            full((1, H)),
            full((H, 4 * LH)),
            full((LH, 4 * LH)),
            full((1, 4 * LH)),
            full((LH, fc_W.shape[0])),
            full((1, fc_W.shape[0])),
        ],
        out_specs=pl.BlockSpec((B, fc_W.shape[0]), lambda w: (0, 0)),
        out_shape=jax.ShapeDtypeStruct((B, fc_W.shape[0]), jnp.float32),
        scratch_shapes=[
            pltpu.VMEM((B, LH), jnp.float32),
            pltpu.VMEM((B, LH), jnp.float32),
            pltpu.VMEM((B, LH), jnp.float32),
        ],
        compiler_params=pltpu.CompilerParams(
            dimension_semantics=("arbitrary",)),
    )(Xn, epad, conv1_W, b1, conv2_W, b2, wih, whh, bias, fcw, fcb)
    return out[:, 0]


# speedup vs baseline: 98.4288x; 2.2105x over previous
import functools

import jax
import jax.numpy as jnp
from jax.experimental import pallas as pl
from jax.experimental.pallas import tpu as pltpu


def _fused_step(x_ref, e_ref, w1_ref, b1_ref, w2_ref, b2_ref,
                wih_ref, whh_ref, bias_ref, fcw_ref, fcb_ref,
                out_ref, h_ref, c_ref, m_ref, *, N, NF, B, H, LH, W, CH):
    w = pl.program_id(0)
    f32 = jnp.float32

    @pl.when(w == 0)
    def _init():
        h_ref[...] = jnp.zeros_like(h_ref)
        c_ref[...] = jnp.zeros_like(c_ref)
        m_ref[...] = jnp.full_like(m_ref, -jnp.inf)

    e = e_ref[...]
    erow = e[0:1, :]
    ecol = e[1:2, :]
    Epad = e.shape[1]
    nio = jax.lax.broadcasted_iota(jnp.int32, (N, Epad), 0).astype(f32)
    Oc = (ecol == nio).astype(f32)
    Or = (erow == nio).astype(f32)
    A = jax.lax.dot_general(Oc, Or, (((1,), (1,)), ((), ())),
                            preferred_element_type=f32)
    ri = jax.lax.broadcasted_iota(jnp.int32, (N, N), 0)
    ci = jax.lax.broadcasted_iota(jnp.int32, (N, N), 1)
    eye = (ri == ci).astype(f32)
    Ah = A + eye
    deg = jnp.sum(Ah, axis=1, keepdims=True)
    dinv = jax.lax.rsqrt(deg)
    D = eye * dinv
    An = jnp.dot(jnp.dot(D, Ah, preferred_element_type=f32), D,
                 preferred_element_type=f32)

    w1 = w1_ref[...]
    w2 = w2_ref[...]
    b1 = b1_ref[...]
    b2t = b2_ref[...]
    pool = jnp.full((1, N), 1.0 / N, dtype=f32)

    for t in range(CH):
        x = x_ref[t]
        ax = jnp.dot(An, x, preferred_element_type=f32)
        ax3 = ax.reshape(N, B, NF)
        acc = ax3[:, :, 0:1] * w1[0].reshape(1, 1, H)
        for k in range(1, NF):
            acc = acc + ax3[:, :, k:k + 1] * w1[k].reshape(1, 1, H)
        xw1 = acc.reshape(N * B, H)
        h1 = jnp.maximum(xw1 + b1, 0.0)
        xw2 = jnp.dot(h1, w2, preferred_element_type=f32)
        t2 = jnp.dot(An, xw2.reshape(N, B * H), preferred_element_type=f32)
        h2 = jnp.maximum(t2 + b2t, 0.0)
        pooled = jnp.dot(pool, h2,
                         preferred_element_type=f32).reshape(B, H)

        gates = (jnp.dot(pooled, wih_ref[...], preferred_element_type=f32)
                 + jnp.dot(h_ref[...], whh_ref[...], preferred_element_type=f32)
                 + bias_ref[...])
        ig = jax.nn.sigmoid(gates[:, 0 * LH:1 * LH])
        fg = jax.nn.sigmoid(gates[:, 1 * LH:2 * LH])
        gg = jnp.tanh(gates[:, 2 * LH:3 * LH])
        og = jax.nn.sigmoid(gates[:, 3 * LH:4 * LH])
        c_new = fg * c_ref[...] + ig * gg
        h_new = og * jnp.tanh(c_new)
        c_ref[...] = c_new
        h_ref[...] = h_new
        m_ref[...] = jnp.maximum(m_ref[...], h_new)

    @pl.when(w == (W // CH) - 1)
    def _final():
        logits = jnp.dot(m_ref[...], fcw_ref[...],
                         preferred_element_type=f32) + fcb_ref[...]
        out_ref[...] = jax.nn.sigmoid(logits)


def kernel(all_keypoint_batches, edge_index, conv1_W, conv1_b, conv2_W,
           conv2_b, W_ih, W_hh, b_ih, b_hh, fc_W, fc_b):
    B, W, N, NF = all_keypoint_batches.shape
    H = conv1_W.shape[1]
    LH = W_hh.shape[1]
    E = edge_index.shape[1]
    CH = 5 if W % 5 == 0 else 1

    Xn = jnp.transpose(all_keypoint_batches, (1, 2, 0, 3)).reshape(W, N, B * NF)

    Epad = max(32, ((E + 31) // 32) * 32)
    epad = jnp.full((8, Epad), -1.0, dtype=jnp.float32)
    epad = epad.at[:2, :E].set(edge_index.astype(jnp.float32))

    wih = W_ih.T
    whh = W_hh.T
    bias = (b_ih + b_hh).reshape(1, 4 * LH)
    b1 = conv1_b.reshape(1, H)
    b2 = jnp.tile(conv2_b, B).reshape(1, B * H)
    fcw = fc_W.T
    fcb = fc_b.reshape(1, fc_W.shape[0])

    full = lambda shape: pl.BlockSpec(shape, lambda w: (0,) * len(shape))
    grid = W // CH

    out = pl.pallas_call(
        functools.partial(_fused_step, N=N, NF=NF, B=B, H=H, LH=LH, W=W,
                          CH=CH),
        grid=(grid,),
        in_specs=[
            pl.BlockSpec((CH, N, B * NF), lambda w: (w, 0, 0)),
            full((8, Epad)),
            full((NF, H)),
            full((1, H)),
            full((H, H)),
            full((1, B * H)),
            full((H, 4 * LH)),
            full((LH, 4 * LH)),
            full((1, 4 * LH)),
            full((LH, fc_W.shape[0])),
            full((1, fc_W.shape[0])),
        ],
        out_specs=pl.BlockSpec((B, fc_W.shape[0]), lambda w: (0, 0)),
        out_shape=jax.ShapeDtypeStruct((B, fc_W.shape[0]), jnp.float32),
        scratch_shapes=[
            pltpu.VMEM((B, LH), jnp.float32),
            pltpu.VMEM((B, LH), jnp.float32),
            pltpu.VMEM((B, LH), jnp.float32),
        ],
        compiler_params=pltpu.CompilerParams(
            dimension_semantics=("arbitrary",)),
    )(Xn, epad, conv1_W, b1, conv2_W, b2, wih, whh, bias, fcw, fcb)
    return out[:, 0]
